# baseline (speedup 1.0000x reference)
import functools

import jax
import jax.numpy as jnp
from jax import lax
from jax.experimental import pallas as pl
from jax.experimental.pallas import tpu as pltpu
from jax.experimental.pallas import tpu_sc as plsc

V_BLK = 4096


def _gather_sc(emb_table, idx):
    batch = idx.shape[0]
    embed = emb_table.shape[1]
    info = plsc.get_sparse_core_info()
    nc, ns = info.num_cores, info.num_subcores
    nw = nc * ns
    b_per_w = batch // nw
    mesh = plsc.VectorSubcoreMesh(core_axis_name="c", subcore_axis_name="s")

    @functools.partial(
        pl.kernel,
        mesh=mesh,
        compiler_params=pltpu.CompilerParams(use_tc_tiling_on_sc=False),
        out_type=jax.ShapeDtypeStruct((batch, embed), jnp.float32),
        scratch_types=[
            pltpu.VMEM((b_per_w,), jnp.int32),
            pltpu.VMEM((b_per_w, embed), jnp.float32),
            pltpu.SemaphoreType.DMA,
        ],
    )
    def gather_kernel(table_hbm, idx_hbm, out_hbm, idx_v, rows_v, sem):
        wid = lax.axis_index("s") * nc + lax.axis_index("c")
        base = wid * b_per_w
        pltpu.sync_copy(idx_hbm.at[pl.ds(base, b_per_w)], idx_v)
        pltpu.async_copy(table_hbm.at[idx_v], rows_v, sem).wait()
        pltpu.sync_copy(rows_v, out_hbm.at[pl.ds(base, b_per_w)])

    return gather_kernel(emb_table, idx)


def _proj_kernel(e_ref, w_ref, b_ref, o_ref):
    o_ref[...] = lax.dot_general(
        e_ref[...], w_ref[...], (((1,), (1,)), ((), ())),
        preferred_element_type=jnp.float32,
    ) + b_ref[...]


def _project(embeds, W, b2):
    batch, embed = embeds.shape
    vocab = W.shape[0]
    return pl.pallas_call(
        _proj_kernel,
        grid=(pl.cdiv(vocab, V_BLK),),
        in_specs=[
            pl.BlockSpec((batch, embed), lambda j: (0, 0)),
            pl.BlockSpec((V_BLK, embed), lambda j: (j, 0)),
            pl.BlockSpec((1, V_BLK), lambda j: (0, j)),
        ],
        out_specs=pl.BlockSpec((batch, V_BLK), lambda j: (0, j)),
        out_shape=jax.ShapeDtypeStruct((batch, vocab), jnp.float32),
    )(embeds, W, b2)


def kernel(x, emb_table, W, b):
    idx = x.reshape(-1).astype(jnp.int32)
    embeds = _gather_sc(emb_table, idx)
    return _project(embeds, W, b.reshape(1, -1))


# speedup vs baseline: 1.0036x; 1.0036x over previous
import functools

import jax
import jax.numpy as jnp
from jax import lax
from jax.experimental import pallas as pl
from jax.experimental.pallas import tpu as pltpu
from jax.experimental.pallas import tpu_sc as plsc

V_BLK = 2048


def _gather_sc(emb_table, idx):
    batch = idx.shape[0]
    embed = emb_table.shape[1]
    info = plsc.get_sparse_core_info()
    nc, ns = info.num_cores, info.num_subcores
    nw = nc * ns
    b_per_w = batch // nw
    mesh = plsc.VectorSubcoreMesh(core_axis_name="c", subcore_axis_name="s")

    @functools.partial(
        pl.kernel,
        mesh=mesh,
        compiler_params=pltpu.CompilerParams(use_tc_tiling_on_sc=False),
        out_type=jax.ShapeDtypeStruct((batch, embed), jnp.float32),
        scratch_types=[
            pltpu.VMEM((b_per_w,), jnp.int32),
            pltpu.VMEM((b_per_w, embed), jnp.float32),
            pltpu.SemaphoreType.DMA,
        ],
    )
    def gather_kernel(table_hbm, idx_hbm, out_hbm, idx_v, rows_v, sem):
        wid = lax.axis_index("s") * nc + lax.axis_index("c")
        base = wid * b_per_w
        pltpu.sync_copy(idx_hbm.at[pl.ds(base, b_per_w)], idx_v)
        pltpu.async_copy(table_hbm.at[idx_v], rows_v, sem).wait()
        pltpu.sync_copy(rows_v, out_hbm.at[pl.ds(base, b_per_w)])

    return gather_kernel(emb_table, idx)


def _proj_kernel(e_ref, w_ref, b_ref, o_ref):
    o_ref[...] = lax.dot_general(
        e_ref[...], w_ref[...], (((1,), (1,)), ((), ())),
        preferred_element_type=jnp.float32,
    ) + b_ref[...]


def _project(embeds, W, b2):
    batch, embed = embeds.shape
    vocab = W.shape[0]
    return pl.pallas_call(
        _proj_kernel,
        grid=(pl.cdiv(vocab, V_BLK),),
        in_specs=[
            pl.BlockSpec((batch, embed), lambda j: (0, 0)),
            pl.BlockSpec((V_BLK, embed), lambda j: (j, 0)),
            pl.BlockSpec((1, V_BLK), lambda j: (0, j)),
        ],
        out_specs=pl.BlockSpec((batch, V_BLK), lambda j: (0, j)),
        out_shape=jax.ShapeDtypeStruct((batch, vocab), jnp.float32),
    )(embeds, W, b2)


def kernel(x, emb_table, W, b):
    idx = x.reshape(-1).astype(jnp.int32)
    embeds = _gather_sc(emb_table, idx)
    return _project(embeds, W, b.reshape(1, -1))
